# Initial kernel scaffold; baseline (speedup 1.0000x reference)
import jax
import jax.numpy as jnp
from jax.experimental import pallas as pl


def kernel(x, tables):
    raise NotImplementedError("write your pallas kernel here")


# speedup vs baseline: 1.0464x; 1.0464x over previous
import functools

import jax
import jax.numpy as jnp
from jax import lax
from jax.experimental import pallas as pl
from jax.experimental.pallas import tpu as pltpu
from jax.experimental.pallas import tpu_sc as plsc

N_FIELDS = 26
VOCAB = 100000
EMBED_DIM = 64
BATCH = 4096

_NC = 2
_NS = 16
_NW = _NC * _NS
_ROWS = BATCH * N_FIELDS
_RPW = _ROWS // _NW
_CHUNK = 128
_NCHUNK = _RPW // _CHUNK
_LANES = 16


@functools.partial(
    pl.kernel,
    out_type=jax.ShapeDtypeStruct((_ROWS, EMBED_DIM), jnp.float32),
    mesh=plsc.VectorSubcoreMesh(core_axis_name="c", subcore_axis_name="s"),
    scratch_types=[
        pltpu.VMEM((_RPW,), jnp.int32),
        pltpu.VMEM((2, _CHUNK, EMBED_DIM), jnp.float32),
        pltpu.SemaphoreType.DMA,
        pltpu.SemaphoreType.DMA,
    ],
    compiler_params=pltpu.CompilerParams(use_tc_tiling_on_sc=False),
)
def _gather(x_hbm, tab_hbm, out_hbm, idx_v, rows_v, sem0, sem1):
    wid = lax.axis_index("s") * _NC + lax.axis_index("c")
    base = wid * _RPW

    pltpu.sync_copy(x_hbm.at[pl.ds(base, _RPW)], idx_v)

    lanes = lax.iota(jnp.int32, _LANES)

    def off_body(t, carry):
        start = t * _LANES
        f = lax.rem(start + lanes, N_FIELDS)
        idx_v[pl.ds(start, _LANES)] = idx_v[pl.ds(start, _LANES)] + f * VOCAB
        return carry

    lax.fori_loop(0, _RPW // _LANES, off_body, 0)

    sems = (sem0, sem1)
    copies = [None, None]
    for j in range(_NCHUNK):
        buf = j % 2
        if copies[buf] is not None:
            copies[buf].wait()
            pltpu.sync_copy(
                rows_v.at[buf],
                out_hbm.at[pl.ds(base + (j - 2) * _CHUNK, _CHUNK)],
            )
        copies[buf] = pltpu.async_copy(
            tab_hbm.at[idx_v.at[pl.ds(j * _CHUNK, _CHUNK)]],
            rows_v.at[buf],
            sems[buf],
        )
    for j in range(_NCHUNK - 2, _NCHUNK):
        buf = j % 2
        copies[buf].wait()
        pltpu.sync_copy(
            rows_v.at[buf],
            out_hbm.at[pl.ds(base + j * _CHUNK, _CHUNK)],
        )


def kernel(x, tables):
    x_flat = x.astype(jnp.int32).reshape(_ROWS)
    tab = tables.reshape(N_FIELDS * VOCAB, EMBED_DIM)
    out = _gather(x_flat, tab)
    return out.reshape(BATCH, N_FIELDS * EMBED_DIM)


# speedup vs baseline: 2.7672x; 2.6445x over previous
import functools

import jax
import jax.numpy as jnp
from jax import lax
from jax.experimental import pallas as pl
from jax.experimental.pallas import tpu as pltpu
from jax.experimental.pallas import tpu_sc as plsc

N_FIELDS = 26
VOCAB = 100000
EMBED_DIM = 64
BATCH = 4096

_NC = 2
_NS = 16
_NW = _NC * _NS
_ROWS = BATCH * N_FIELDS
_RPW = _ROWS // _NW
_CHUNK = 32
_NCHUNK = _RPW // _CHUNK
_NBUF = 4
_LANES = 16
_TILES = N_FIELDS * VOCAB // 8


@functools.partial(
    pl.kernel,
    out_type=jax.ShapeDtypeStruct((_ROWS, EMBED_DIM), jnp.float32),
    mesh=plsc.VectorSubcoreMesh(core_axis_name="c", subcore_axis_name="s"),
    scratch_types=[
        pltpu.VMEM((_RPW,), jnp.int32),
        pltpu.VMEM((_CHUNK, EMBED_DIM), jnp.float32),
        pltpu.VMEM((_CHUNK, EMBED_DIM), jnp.float32),
        pltpu.VMEM((_CHUNK, EMBED_DIM), jnp.float32),
        pltpu.VMEM((_CHUNK, EMBED_DIM), jnp.float32),
        pltpu.SemaphoreType.DMA,
        pltpu.SemaphoreType.DMA,
        pltpu.SemaphoreType.DMA,
        pltpu.SemaphoreType.DMA,
        pltpu.SemaphoreType.DMA,
        pltpu.SemaphoreType.DMA,
        pltpu.SemaphoreType.DMA,
        pltpu.SemaphoreType.DMA,
    ],
    compiler_params=pltpu.CompilerParams(use_tc_tiling_on_sc=True),
)
def _gather(x_hbm, tab_hbm, out_hbm, xidx, r0, r1, r2, r3,
            g0, g1, g2, g3, o0, o1, o2, o3):
    wid = lax.axis_index("s") * _NC + lax.axis_index("c")
    base = wid * _RPW
    rbufs = (r0, r1, r2, r3)
    gsems = (g0, g1, g2, g3)
    osems = (o0, o1, o2, o3)
    lanes = lax.iota(jnp.int32, _LANES)

    pltpu.sync_copy(x_hbm.at[pl.ds(base, _RPW)], xidx)

    def split_body(t, carry):
        st = t * _LANES
        xidx[pl.ds(st, _LANES)] = (
            xidx[pl.ds(st, _LANES)] + lax.rem(st + lanes, N_FIELDS) * VOCAB)
        return carry

    lax.fori_loop(0, _RPW // _LANES, split_body, 0)

    def fire(m, b):
        for grp in range(_CHUNK // _LANES):
            gv = xidx[pl.ds(m * _CHUNK + grp * _LANES, _LANES)]
            tv = lax.shift_right_logical(gv, 3)
            sv = lax.bitwise_and(gv, 7)
            for j in range(_LANES):
                pltpu.make_async_copy(
                    tab_hbm.at[tv[j], pl.ds(sv[j], 1)],
                    rbufs[b].at[pl.ds(grp * _LANES + j, 1)],
                    gsems[b]).start()

    def gwait(b):
        pltpu.make_async_copy(
            out_hbm.at[pl.ds(0, _CHUNK)], rbufs[b], gsems[b]).wait()

    def ostart(m, b):
        pltpu.make_async_copy(
            rbufs[b], out_hbm.at[pl.ds(base + m * _CHUNK, _CHUNK)],
            osems[b]).start()

    def owait(b):
        pltpu.make_async_copy(
            rbufs[b], out_hbm.at[pl.ds(base, _CHUNK)], osems[b]).wait()

    for m in range(_NBUF):
        fire(m, m)

    def pipe_body(i, carry):
        for b in range(_NBUF):
            m = _NBUF * i + b
            gwait(b)
            ostart(m, b)
            owait(b)
            fire(m + _NBUF, b)
        return carry

    lax.fori_loop(0, _NCHUNK // _NBUF - 1, pipe_body, 0)

    for m in range(_NCHUNK - _NBUF, _NCHUNK):
        b = m % _NBUF
        gwait(b)
        ostart(m, b)
        owait(b)


def kernel(x, tables):
    x_flat = x.astype(jnp.int32).reshape(_ROWS)
    tab = tables.reshape(_TILES, 8, EMBED_DIM)
    out = _gather(x_flat, tab)
    return out.reshape(BATCH, N_FIELDS * EMBED_DIM)
